# baseline (speedup 1.0000x reference)
import functools

import jax
import jax.numpy as jnp
from jax import lax
from jax.experimental import pallas as pl
from jax.experimental.pallas import tpu as pltpu
from jax.experimental.pallas import tpu_sc as plsc

NUM_SUBCORES = 16
NUM_CORES = 2
NW = NUM_CORES * NUM_SUBCORES

CHUNK = 128
NBUF = 5


@functools.cache
def _build(n_rows, vocab, d):
    n_chunks = n_rows // NW
    groups = n_chunks // NBUF
    mesh = plsc.VectorSubcoreMesh(core_axis_name="c", subcore_axis_name="s")

    def body(idx_hbm, table_hbm, out_hbm, idx_v, *rest):
        bufs = rest[:NBUF]
        gsems = rest[NBUF:2 * NBUF]
        ssems = rest[2 * NBUF:]

        wid = lax.axis_index("c") * NUM_SUBCORES + lax.axis_index("s")
        id0 = wid * n_chunks * CHUNK

        pltpu.sync_copy(idx_hbm.at[pl.ds(id0, n_chunks * CHUNK)], idx_v)

        def start_gather(j, b):
            pltpu.async_copy(
                table_hbm.at[idx_v.at[pl.ds(j * CHUNK, CHUNK)]],
                bufs[b], gsems[b])

        def wait_gather(j, b):
            pltpu.make_async_copy(
                table_hbm.at[idx_v.at[pl.ds(j * CHUNK, CHUNK)]],
                bufs[b], gsems[b]).wait()

        def start_scatter(j, b):
            pltpu.async_copy(
                bufs[b], out_hbm.at[pl.ds(id0 + j * CHUNK, CHUNK)], ssems[b])

        def wait_scatter(j, b):
            pltpu.make_async_copy(
                bufs[b], out_hbm.at[pl.ds(id0 + j * CHUNK, CHUNK)],
                ssems[b]).wait()

        for b in range(NBUF):
            start_gather(b, b)

        @pl.loop(0, groups - 1)
        def _(g):
            for b in range(NBUF):
                j = g * NBUF + b
                wait_gather(j, b)
                start_scatter(j, b)
            for b in range(NBUF):
                j = g * NBUF + b
                wait_scatter(j, b)
                start_gather(j + NBUF, b)

        for b in range(NBUF):
            j = (groups - 1) * NBUF + b
            wait_gather(j, b)
            start_scatter(j, b)
        for b in range(NBUF):
            j = (groups - 1) * NBUF + b
            wait_scatter(j, b)

    run = pl.kernel(
        body,
        out_type=jax.ShapeDtypeStruct((n_rows * CHUNK, d), jnp.float32),
        mesh=mesh,
        scratch_types=(
            [pltpu.VMEM((n_chunks * CHUNK,), jnp.int32)]
            + [pltpu.VMEM((CHUNK, d), jnp.float32) for _ in range(NBUF)]
            + [pltpu.SemaphoreType.DMA for _ in range(2 * NBUF)]
        ),
    )
    return run


def kernel(token_ids, weight):
    bsz, seq = token_ids.shape
    vocab, d = weight.shape
    total = bsz * seq
    idx1d = token_ids.astype(jnp.int32).reshape(total)
    out = _build(total // CHUNK, vocab, d)(idx1d, weight)
    return out.reshape(bsz, seq, d)


# speedup vs baseline: 1.7715x; 1.7715x over previous
import functools

import jax
import jax.numpy as jnp
from jax import lax
from jax.experimental import pallas as pl
from jax.experimental.pallas import tpu as pltpu
from jax.experimental.pallas import tpu_sc as plsc

NUM_SUBCORES = 16
NUM_CORES = 2
NW = NUM_CORES * NUM_SUBCORES

KB = 2
NBUF = 4


@functools.cache
def _build(bsz, seq, vocab, d):
    nb = bsz // NW
    n_chunks = nb // KB
    groups = n_chunks // NBUF
    ids_per_chunk = KB * seq
    mesh = plsc.VectorSubcoreMesh(core_axis_name="c", subcore_axis_name="s")

    def body(idx_hbm, table_hbm, out_hbm, idx_v, *rest):
        bufs = rest[:NBUF]
        gsems = rest[NBUF:2 * NBUF]
        ssems = rest[2 * NBUF:]

        wid = lax.axis_index("c") * NUM_SUBCORES + lax.axis_index("s")
        b0 = wid * nb

        pltpu.sync_copy(idx_hbm.at[pl.ds(wid * n_chunks, n_chunks)], idx_v)

        def start_gather(j, b):
            pltpu.async_copy(table_hbm.at[idx_v.at[j]], bufs[b], gsems[b])

        def wait_gather(j, b):
            pltpu.make_async_copy(
                table_hbm.at[idx_v.at[j]], bufs[b], gsems[b]).wait()

        def start_scatter(j, b):
            for k in range(KB):
                pltpu.async_copy(
                    bufs[b].at[pl.ds(k * seq, seq)],
                    out_hbm.at[b0 + j * KB + k], ssems[b])

        def wait_scatter(j, b):
            for k in range(KB):
                pltpu.make_async_copy(
                    bufs[b].at[pl.ds(k * seq, seq)],
                    out_hbm.at[b0 + j * KB + k], ssems[b]).wait()

        for b in range(NBUF):
            start_gather(b, b)

        @pl.loop(0, groups - 1)
        def _(g):
            for b in range(NBUF):
                j = g * NBUF + b
                wait_gather(j, b)
                start_scatter(j, b)
            for b in range(NBUF):
                j = g * NBUF + b
                wait_scatter(j, b)
                start_gather(j + NBUF, b)

        for b in range(NBUF):
            j = (groups - 1) * NBUF + b
            wait_gather(j, b)
            start_scatter(j, b)
        for b in range(NBUF):
            j = (groups - 1) * NBUF + b
            wait_scatter(j, b)

    run = pl.kernel(
        body,
        out_type=jax.ShapeDtypeStruct((bsz, seq, d), jnp.float32),
        mesh=mesh,
        scratch_types=(
            [pltpu.VMEM((n_chunks, ids_per_chunk), jnp.int32)]
            + [pltpu.VMEM((ids_per_chunk, d), jnp.float32) for _ in range(NBUF)]
            + [pltpu.SemaphoreType.DMA for _ in range(2 * NBUF)]
        ),
    )
    return run


def kernel(token_ids, weight):
    bsz, seq = token_ids.shape
    vocab, d = weight.shape
    idx2d = token_ids.astype(jnp.int32).reshape(bsz * seq // (KB * seq), KB * seq)
    return _build(bsz, seq, vocab, d)(idx2d, weight)


# speedup vs baseline: 3.1295x; 1.7666x over previous
import functools

import jax
import jax.numpy as jnp
from jax import lax
from jax.experimental import pallas as pl
from jax.experimental.pallas import tpu as pltpu
from jax.experimental.pallas import tpu_sc as plsc

NUM_SUBCORES = 16
NUM_CORES = 2
NW = NUM_CORES * NUM_SUBCORES

NBUF = 5


@functools.cache
def _build(bsz, seq, vocab, d):
    nb = bsz // NW
    n_chunks = seq
    groups = n_chunks // NBUF
    mesh = plsc.VectorSubcoreMesh(core_axis_name="c", subcore_axis_name="s")

    def body(idx_hbm, table_hbm, out_hbm, idx_v, *rest):
        bufs = rest[:NBUF]
        gsems = rest[NBUF:2 * NBUF]
        ssems = rest[2 * NBUF:]

        wid = lax.axis_index("c") * NUM_SUBCORES + lax.axis_index("s")
        b0 = wid * nb

        pltpu.sync_copy(idx_hbm.at[:, pl.ds(b0, nb)], idx_v)

        def start_gather(t, b):
            pltpu.async_copy(table_hbm.at[idx_v.at[t]], bufs[b], gsems[b])

        def wait_gather(t, b):
            pltpu.make_async_copy(
                table_hbm.at[idx_v.at[t]], bufs[b], gsems[b]).wait()

        def start_scatter(t, b):
            pltpu.async_copy(
                bufs[b], out_hbm.at[t, pl.ds(b0, nb)], ssems[b])

        def wait_scatter(t, b):
            pltpu.make_async_copy(
                bufs[b], out_hbm.at[t, pl.ds(b0, nb)], ssems[b]).wait()

        for b in range(NBUF):
            start_gather(b, b)

        @pl.loop(0, groups - 1)
        def _(g):
            for b in range(NBUF):
                t = g * NBUF + b
                wait_gather(t, b)
                start_scatter(t, b)
            for b in range(NBUF):
                t = g * NBUF + b
                wait_scatter(t, b)
                start_gather(t + NBUF, b)

        for b in range(NBUF):
            t = (groups - 1) * NBUF + b
            wait_gather(t, b)
            start_scatter(t, b)
        for b in range(NBUF):
            t = (groups - 1) * NBUF + b
            wait_scatter(t, b)

    run = pl.kernel(
        body,
        out_type=jax.ShapeDtypeStruct((seq, bsz, d), jnp.float32),
        mesh=mesh,
        scratch_types=(
            [pltpu.VMEM((seq, nb), jnp.int32)]
            + [pltpu.VMEM((nb, d), jnp.float32) for _ in range(NBUF)]
            + [pltpu.SemaphoreType.DMA for _ in range(2 * NBUF)]
        ),
    )
    return run


def kernel(token_ids, weight):
    bsz, seq = token_ids.shape
    vocab, d = weight.shape
    idx_t = token_ids.astype(jnp.int32).T
    out = _build(bsz, seq, vocab, d)(idx_t, weight)
    return out.transpose(1, 0, 2)


# speedup vs baseline: 3.2113x; 1.0261x over previous
import functools

import jax
import jax.numpy as jnp
from jax import lax
from jax.experimental import pallas as pl
from jax.experimental.pallas import tpu as pltpu
from jax.experimental.pallas import tpu_sc as plsc

NUM_SUBCORES = 16
NUM_CORES = 2
NW = NUM_CORES * NUM_SUBCORES

NBUF = 10
HALVES = 2


@functools.cache
def _build(bsz, seq, vocab, d):
    nb = bsz // NW
    hw = nb // HALVES
    n_chunks = seq * HALVES
    groups = n_chunks // NBUF
    mesh = plsc.VectorSubcoreMesh(core_axis_name="c", subcore_axis_name="s")

    def body(idx_hbm, table_hbm, out_hbm, idx_v, *rest):
        bufs = rest[:NBUF]
        gsems = rest[NBUF:2 * NBUF]
        ssems = rest[2 * NBUF:]

        wid = lax.axis_index("c") * NUM_SUBCORES + lax.axis_index("s")
        b0 = wid * nb

        pltpu.sync_copy(idx_hbm.at[:, pl.ds(b0, nb)], idx_v)

        def _idx(c):
            t, h = c // HALVES, c % HALVES
            return idx_v.at[t, pl.ds(h * hw, hw)]

        def _dst(c):
            t, h = c // HALVES, c % HALVES
            return out_hbm.at[t, pl.ds(b0 + h * hw, hw)]

        def start_gather(c, b):
            pltpu.async_copy(table_hbm.at[_idx(c)], bufs[b], gsems[b])

        def wait_gather(c, b):
            pltpu.make_async_copy(table_hbm.at[_idx(c)], bufs[b], gsems[b]).wait()

        def start_scatter(c, b):
            pltpu.async_copy(bufs[b], _dst(c), ssems[b])

        def wait_scatter(c, b):
            pltpu.make_async_copy(bufs[b], _dst(c), ssems[b]).wait()

        for b in range(NBUF):
            start_gather(b, b)

        @pl.loop(0, groups - 1)
        def _(g):
            for b in range(NBUF):
                c = g * NBUF + b
                wait_gather(c, b)
                start_scatter(c, b)
            for b in range(NBUF):
                c = g * NBUF + b
                wait_scatter(c, b)
                start_gather(c + NBUF, b)

        for b in range(NBUF):
            c = (groups - 1) * NBUF + b
            wait_gather(c, b)
            start_scatter(c, b)
        for b in range(NBUF):
            c = (groups - 1) * NBUF + b
            wait_scatter(c, b)

    run = pl.kernel(
        body,
        out_type=jax.ShapeDtypeStruct((seq, bsz, d), jnp.float32),
        mesh=mesh,
        scratch_types=(
            [pltpu.VMEM((seq, nb), jnp.int32)]
            + [pltpu.VMEM((hw, d), jnp.float32) for _ in range(NBUF)]
            + [pltpu.SemaphoreType.DMA for _ in range(2 * NBUF)]
        ),
    )
    return run


def kernel(token_ids, weight):
    bsz, seq = token_ids.shape
    vocab, d = weight.shape
    idx_t = token_ids.astype(jnp.int32).T
    out = _build(bsz, seq, vocab, d)(idx_t, weight)
    return out.transpose(1, 0, 2)
